# baseline (speedup 1.0000x reference)
import functools

import jax
import jax.numpy as jnp
from jax import lax
from jax.experimental import pallas as pl
from jax.experimental.pallas import tpu as pltpu
from jax.experimental.pallas import tpu_sc as plsc

_NSUB = 16
_NCORE = 2
_L = 16


def _sc_order_body(C, vm_hbm, order_hbm, vrow, rall, st, shared):
    b = lax.axis_index("c")
    sid = lax.axis_index("s")
    nchunk = C // _L
    tA = sid
    tB = _NSUB + sid
    has_b = sid < (nchunk - _NSUB)

    pltpu.sync_copy(vm_hbm.at[pl.ds(b * C, C)], vrow.at[pl.ds(_L, C)])
    iota = lax.broadcasted_iota(jnp.int32, (_L,), 0)
    zvec = iota - iota
    vA = vrow[pl.ds(_L + _L * tA, _L)]
    vB = vrow[pl.ds(_L + _L * tB, _L)]
    cA = _L * tA + iota
    cB = _L * tB + iota

    def rstep(o, carry):
        rankA, rankB = carry
        w = vrow[pl.ds(o, _L)]
        cp = o + iota - _L
        valid = (cp >= 0) & (cp < C)
        bA = valid & ((w > vA) | ((w == vA) & (cp < cA)))
        bB = valid & ((w > vB) | ((w == vB) & (cp < cB)))
        one = jnp.where(bA, 1, 0).astype(jnp.int32)
        two = jnp.where(bB, 1, 0).astype(jnp.int32)
        return rankA + one, rankB + two

    rankA, rankB = lax.fori_loop(0, C + _L, rstep, (zvec, zvec))
    st[pl.ds(0, _L)] = rankA
    st[pl.ds(_L, _L)] = rankB
    pltpu.sync_copy(st.at[pl.ds(0, _L)], shared.at[pl.ds(_L * tA, _L)])

    @pl.when(has_b)
    def _stage_b():
        pltpu.sync_copy(st.at[pl.ds(_L, _L)], shared.at[pl.ds(_L * tB, _L)])

    plsc.subcore_barrier()

    pltpu.sync_copy(shared, rall.at[pl.ds(_L, C)])
    jA = cA
    jB = cB

    def ostep(o, carry):
        ordA, ordB = carry
        wr = rall[pl.ds(o, _L)]
        cp = o + iota - _L
        valid = (cp >= 0) & (cp < C)
        mA = valid & (wr == jA)
        mB = valid & (wr == jB)
        return (ordA + jnp.where(mA, cp, 0), ordB + jnp.where(mB, cp, 0))

    ordA, ordB = lax.fori_loop(0, C + _L, ostep, (zvec, zvec))
    st[pl.ds(0, _L)] = ordA
    st[pl.ds(_L, _L)] = ordB
    pltpu.sync_copy(st.at[pl.ds(0, _L)],
                    order_hbm.at[pl.ds(b * C + _L * tA, _L)])

    @pl.when(has_b)
    def _out_b():
        pltpu.sync_copy(st.at[pl.ds(_L, _L)],
                        order_hbm.at[pl.ds(b * C + _L * tB, _L)])


def _permute_body(JC, nex_ref, x_ref, g_ref, o_ref):
    M = x_ref.shape[1] * x_ref.shape[2] * x_ref.shape[3]
    C = x_ref.shape[4]
    nex = nex_ref[0]
    xm = x_ref[...].reshape(M, C)
    NT = C // JC
    for t in range(NT):
        active = t * JC < nex

        @pl.when(active)
        def _gather(t=t):
            idxt = jnp.broadcast_to(
                g_ref[0, 0, t * JC:(t + 1) * JC].reshape(1, JC), (M, JC)
            )
            local = jnp.bitwise_and(idxt, JC - 1)
            tile = jnp.right_shift(idxt, 7)
            acc = jnp.zeros((M, JC), jnp.float32)
            for s_tile in range(NT):
                xs = xm[:, s_tile * JC:(s_tile + 1) * JC]
                gs = jnp.take_along_axis(xs, local, axis=1)
                acc = jnp.where(tile == s_tile, gs, acc)
            jvec = jax.lax.broadcasted_iota(jnp.int32, (M, JC), 1) + t * JC
            o_ref[..., t * JC:(t + 1) * JC] = jnp.where(
                jvec < nex, acc, 0.0
            ).reshape(o_ref.shape[:-1] + (JC,))

        @pl.when(jnp.logical_not(active))
        def _zero(t=t):
            o_ref[..., t * JC:(t + 1) * JC] = jnp.zeros(
                o_ref.shape[:-1] + (JC,), o_ref.dtype
            )


def kernel(x, exist_ratio):
    B, C, D, H, W = x.shape
    c_hi = float(1 << (C.bit_length() - 1))
    c_lo = float(C) - c_hi

    val_mean = jnp.mean(jnp.abs(x), axis=(2, 3, 4))

    mesh = plsc.VectorSubcoreMesh(
        core_axis_name="c", subcore_axis_name="s",
        num_cores=_NCORE, num_subcores=_NSUB,
    )
    order_flat = pl.kernel(
        functools.partial(_sc_order_body, C),
        out_type=jax.ShapeDtypeStruct((B * C,), jnp.int32),
        mesh=mesh,
        scratch_types=[
            pltpu.VMEM((C + 2 * _L,), jnp.float32),
            pltpu.VMEM((C + 2 * _L,), jnp.int32),
            pltpu.VMEM((2 * _L,), jnp.int32),
            pltpu.VMEM_SHARED((C,), jnp.int32),
        ],
    )(val_mean.reshape(B * C))

    rvs = exist_ratio.astype(jnp.float32)
    hi = rvs * jnp.float32(c_hi)
    lo = rvs * jnp.float32(c_lo)
    s = hi + lo
    err = lo - (s - hi)
    n = jnp.floor(s)
    frac = (s - n) + err
    nexi = (n + jnp.floor(frac)).astype(jnp.int32).reshape(1)

    gidx3 = order_flat.reshape(B, 1, C)
    y = jnp.transpose(x, (0, 2, 3, 4, 1))
    JC = 128
    DB = 2
    grid_spec = pltpu.PrefetchScalarGridSpec(
        num_scalar_prefetch=1,
        grid=(B, D // DB),
        in_specs=[
            pl.BlockSpec((1, DB, H, W, C), lambda b, d, nn: (b, d, 0, 0, 0)),
            pl.BlockSpec((1, 1, C), lambda b, d, nn: (b, 0, 0)),
        ],
        out_specs=pl.BlockSpec(
            (1, DB, H, W, C), lambda b, d, nn: (b, d, 0, 0, 0)
        ),
    )
    out_perm = pl.pallas_call(
        functools.partial(_permute_body, JC),
        grid_spec=grid_spec,
        out_shape=jax.ShapeDtypeStruct((B, D, H, W, C), x.dtype),
    )(nexi, y, gidx3)
    return jnp.transpose(out_perm, (0, 4, 1, 2, 3))
